# Initial kernel scaffold; baseline (speedup 1.0000x reference)
import jax
import jax.numpy as jnp
from jax.experimental import pallas as pl


def kernel(x, edge_index, W1, b1, W2, b2, gamma, beta):
    raise NotImplementedError("write your pallas kernel here")


# speedup vs baseline: 7.3112x; 7.3112x over previous
import functools

import jax
import jax.numpy as jnp
from jax import lax
from jax.experimental import pallas as pl
from jax.experimental.pallas import tpu as pltpu
from jax.experimental.pallas import tpu_sc as plsc

N_NODES = 10000
N_EDGES = 320000
D_IN = 128
D_HID = 256

NC = 2
NS = 16
NW = NC * NS
EDGES_PER_TILE = N_EDGES // NW
EB = 80
NCHUNK = EDGES_PER_TILE // EB
WB_TILES = 10
WB_ROWS = N_NODES // WB_TILES


def _sc_agg_body(x_hbm, src_hbm, dst_hbm, parts_hbm,
                 acc, idx_src, idx_dst, rows, sem):
    cid = lax.axis_index("c")
    sid = lax.axis_index("s")
    wid = sid * NC + cid

    @pl.when(sid == 0)
    def _():
        pltpu.sync_copy(x_hbm, acc)

    pltpu.sync_copy(src_hbm.at[wid], idx_src)
    pltpu.sync_copy(dst_hbm.at[wid], idx_dst)
    plsc.subcore_barrier()

    def step(j, carry):
        pltpu.async_copy(x_hbm.at[idx_src.at[j]], rows, sem).wait()
        pltpu.sync_copy(rows, acc.at[idx_dst.at[j]], add=True)
        return carry

    lax.fori_loop(0, NCHUNK, step, 0)
    plsc.subcore_barrier()

    @pl.when(sid < WB_TILES)
    def _():
        pltpu.sync_copy(acc.at[pl.ds(sid * WB_ROWS, WB_ROWS)],
                        parts_hbm.at[cid, pl.ds(sid * WB_ROWS, WB_ROWS)])


@jax.jit
def _sc_agg(x, src, dst):
    src_r = src.reshape(NW, NCHUNK, EB)
    dst_r = dst.reshape(NW, NCHUNK, EB)
    mesh = plsc.VectorSubcoreMesh(core_axis_name="c", subcore_axis_name="s")
    return pl.kernel(
        _sc_agg_body,
        out_type=jax.ShapeDtypeStruct((NC, N_NODES, D_IN), jnp.float32),
        mesh=mesh,
        scratch_types=[
            pltpu.VMEM_SHARED((N_NODES, D_IN), jnp.float32),
            pltpu.VMEM((NCHUNK, EB), jnp.int32),
            pltpu.VMEM((NCHUNK, EB), jnp.int32),
            pltpu.VMEM((EB, D_IN), jnp.float32),
            pltpu.SemaphoreType.DMA,
        ],
    )(x, src_r, dst_r)


ROW_BLK = 1000
N_BLKS = N_NODES // ROW_BLK


def _tc_mlp_bn_body(x_ref, p0_ref, p1_ref, w1_ref, b1_ref, w2_ref, b2_ref,
                    gamma_ref, beta_ref, out_ref, stats_ref):
    i = pl.program_id(0)
    h = p0_ref[...] + p1_ref[...] - x_ref[...]
    h1 = jnp.maximum(
        jnp.dot(h, w1_ref[...], preferred_element_type=jnp.float32) + b1_ref[...],
        0.0)
    h2 = jnp.maximum(
        jnp.dot(h1, w2_ref[...], preferred_element_type=jnp.float32) + b2_ref[...],
        0.0)
    out_ref[pl.ds(i * ROW_BLK, ROW_BLK), :] = h2
    s = jnp.sum(h2, axis=0, keepdims=True)
    q = jnp.sum(h2 * h2, axis=0, keepdims=True)

    @pl.when(i == 0)
    def _():
        stats_ref[0:1, :] = s
        stats_ref[1:2, :] = q

    @pl.when(i > 0)
    def _():
        stats_ref[0:1, :] += s
        stats_ref[1:2, :] += q

    @pl.when(i == N_BLKS - 1)
    def _():
        mean = stats_ref[0:1, :] / N_NODES
        var = stats_ref[1:2, :] / N_NODES - mean * mean
        inv = lax.rsqrt(var + 1e-5) * gamma_ref[...]
        out_ref[...] = (out_ref[...] - mean) * inv + beta_ref[...]


@jax.jit
def _tc_mlp_bn(x, p0, p1, W1, b1, W2, b2, gamma, beta):
    return pl.pallas_call(
        _tc_mlp_bn_body,
        grid=(N_BLKS,),
        in_specs=[
            pl.BlockSpec((ROW_BLK, D_IN), lambda i: (i, 0)),
            pl.BlockSpec((ROW_BLK, D_IN), lambda i: (i, 0)),
            pl.BlockSpec((ROW_BLK, D_IN), lambda i: (i, 0)),
            pl.BlockSpec((D_IN, D_HID), lambda i: (0, 0)),
            pl.BlockSpec((1, D_HID), lambda i: (0, 0)),
            pl.BlockSpec((D_HID, D_HID), lambda i: (0, 0)),
            pl.BlockSpec((1, D_HID), lambda i: (0, 0)),
            pl.BlockSpec((1, D_HID), lambda i: (0, 0)),
            pl.BlockSpec((1, D_HID), lambda i: (0, 0)),
        ],
        out_specs=pl.BlockSpec((N_NODES, D_HID), lambda i: (0, 0)),
        out_shape=jax.ShapeDtypeStruct((N_NODES, D_HID), jnp.float32),
        scratch_shapes=[pltpu.VMEM((2, D_HID), jnp.float32)],
    )(x, p0, p1, W1, b1.reshape(1, -1), W2, b2.reshape(1, -1),
      gamma.reshape(1, -1), beta.reshape(1, -1))


def kernel(x, edge_index, W1, b1, W2, b2, gamma, beta):
    src = edge_index[0].astype(jnp.int32)
    dst = edge_index[1].astype(jnp.int32)
    parts = _sc_agg(x, src, dst)
    return _tc_mlp_bn(x, parts[0], parts[1], W1, b1, W2, b2, gamma, beta)
